# Initial kernel scaffold; baseline (speedup 1.0000x reference)
import jax
import jax.numpy as jnp
from jax.experimental import pallas as pl


def kernel(im, disp):
    raise NotImplementedError("write your pallas kernel here")


# speedup vs baseline: 28.4887x; 28.4887x over previous
import functools

import jax
import jax.numpy as jnp
import numpy as np
from jax import lax
from jax.experimental import pallas as pl
from jax.experimental.pallas import tpu as pltpu
from jax.experimental.pallas import tpu_sc as plsc

_B, _C, _H, _W = 8, 3, 384, 512
_LANES = 16
_GROUPS = (_B * _H) // _LANES
_GROUPS_PER_B = _H // _LANES
_GSZ = _W * _LANES
_IMSZ = _C * _GSZ
_NWORKERS = 32
_GPW = _GROUPS // _NWORKERS
_EPS = 1e-6
_LN_BASE = float(np.log(1.414))
_MAGIC = float(1.5 * (2 ** 23))


def _warp_body(im_hbm, disp_hbm, res_hbm, occ_hbm, disp_v, im_v, racc, macc,
               oacc):
    nc = 2
    wid = lax.axis_index("s") * nc + lax.axis_index("c")

    lane = lax.iota(jnp.int32, _LANES)
    lane_w = lane * _W
    zeros = jnp.zeros((_LANES,), jnp.float32)
    ones = jnp.ones((_LANES,), jnp.float32)

    def zero_body(i, _):
        b = i * _LANES
        macc[pl.ds(b, _LANES)] = zeros
        oacc[pl.ds(b, _LANES)] = zeros
        for c in range(_C):
            racc[pl.ds(c * _GSZ + b, _LANES)] = zeros
        return 0

    lax.fori_loop(0, _W, zero_body, 0)

    def group_body(k, _):
        g = wid * _GPW + k
        b = g // _GROUPS_PER_B
        im_base = b * (_C * _GROUPS_PER_B * _GSZ) + (g % _GROUPS_PER_B) * _GSZ

        pltpu.sync_copy(disp_hbm.at[pl.ds(g * _GSZ, _GSZ)], disp_v)
        for c in range(_C):
            pltpu.sync_copy(
                im_hbm.at[pl.ds(im_base + c * (_GROUPS_PER_B * _GSZ), _GSZ)],
                im_v.at[pl.ds(c * _GSZ, _GSZ)])

        def acc_body(x, _):
            gidx = lane_w + x
            d = plsc.load_gather(disp_v, [gidx])
            xf = jnp.broadcast_to(x, (_LANES,)).astype(jnp.float32)
            t = xf - d
            tr = (t + _MAGIC) - _MAGIC
            valid = tr >= 0.0
            ti = jnp.maximum(tr.astype(jnp.int32), 0)
            w = jnp.exp(d * _LN_BASE)
            didx = ti * _LANES + lane
            plsc.addupdate_scatter(macc, [didx], w, mask=valid)
            plsc.addupdate_scatter(oacc, [didx], ones, mask=valid)
            for c in range(_C):
                v = plsc.load_gather(im_v, [gidx + c * _GSZ]) * w
                plsc.addupdate_scatter(racc, [didx + c * _GSZ], v, mask=valid)
            return 0

        lax.fori_loop(0, _W, acc_body, 0)

        def post_body(t, _):
            base = t * _LANES
            m = macc[pl.ds(base, _LANES)]
            o = oacc[pl.ds(base, _LANES)]
            macc[pl.ds(base, _LANES)] = zeros
            oacc[pl.ds(base, _LANES)] = zeros
            inv = 1.0 / jnp.maximum(m, _EPS)
            out_idx = lane_w + t
            plsc.store_scatter(disp_v, [out_idx], 1.0 - jnp.minimum(o, 1.0))
            for c in range(_C):
                r = racc[pl.ds(c * _GSZ + base, _LANES)]
                racc[pl.ds(c * _GSZ + base, _LANES)] = zeros
                plsc.store_scatter(im_v, [out_idx + c * _GSZ], r * inv)
            return 0

        lax.fori_loop(0, _W, post_body, 0)

        pltpu.sync_copy(disp_v, occ_hbm.at[pl.ds(g * _GSZ, _GSZ)])
        for c in range(_C):
            pltpu.sync_copy(
                im_v.at[pl.ds(c * _GSZ, _GSZ)],
                res_hbm.at[pl.ds(im_base + c * (_GROUPS_PER_B * _GSZ), _GSZ)])
        return 0

    lax.fori_loop(0, _GPW, group_body, 0)


_warp = functools.partial(
    pl.kernel,
    out_type=[
        jax.ShapeDtypeStruct((_B * _C * _H * _W,), jnp.float32),
        jax.ShapeDtypeStruct((_B * _H * _W,), jnp.float32),
    ],
    mesh=plsc.VectorSubcoreMesh(core_axis_name="c", subcore_axis_name="s"),
    compiler_params=pltpu.CompilerParams(needs_layout_passes=False),
    scratch_types=[
        pltpu.VMEM((_GSZ,), jnp.float32),
        pltpu.VMEM((_IMSZ,), jnp.float32),
        pltpu.VMEM((_IMSZ,), jnp.float32),
        pltpu.VMEM((_GSZ,), jnp.float32),
        pltpu.VMEM((_GSZ,), jnp.float32),
    ],
)(_warp_body)


def kernel(im, disp):
    res_flat, occ_flat = _warp(im.reshape(-1), disp.reshape(-1))
    return (res_flat.reshape(_B, _C, _H, _W),
            occ_flat.reshape(_B, 1, _H, _W))


# speedup vs baseline: 28.7666x; 1.0098x over previous
import functools

import jax
import jax.numpy as jnp
import numpy as np
from jax import lax
from jax.experimental import pallas as pl
from jax.experimental.pallas import tpu as pltpu
from jax.experimental.pallas import tpu_sc as plsc

_B, _C, _H, _W = 8, 3, 384, 512
_LANES = 16
_GROUPS = (_B * _H) // _LANES
_GROUPS_PER_B = _H // _LANES
_GSZ = _W * _LANES
_IMSZ = _C * _GSZ
_NWORKERS = 32
_GPW = _GROUPS // _NWORKERS
_EPS = 1e-6
_LN_BASE = float(np.log(1.414))
_MAGIC = float(1.5 * (2 ** 23))


def _warp_body(im_hbm, disp_hbm, res_hbm, occ_hbm, disp_v, im_v, racc, macc,
               oacc):
    nc = 2
    wid = lax.axis_index("s") * nc + lax.axis_index("c")

    lane = lax.iota(jnp.int32, _LANES)
    lane_w = lane * _W
    zeros = jnp.zeros((_LANES,), jnp.float32)
    ones = jnp.ones((_LANES,), jnp.float32)

    def zero_body(i, _):
        for u in range(4):
            b = (i * 4 + u) * _LANES
            macc[pl.ds(b, _LANES)] = zeros
            oacc[pl.ds(b, _LANES)] = zeros
            for c in range(_C):
                racc[pl.ds(c * _GSZ + b, _LANES)] = zeros
        return 0

    lax.fori_loop(0, _W // 4, zero_body, 0)

    def group_body(k, _):
        g = wid * _GPW + k
        b = g // _GROUPS_PER_B
        im_base = b * (_C * _GROUPS_PER_B * _GSZ) + (g % _GROUPS_PER_B) * _GSZ

        pltpu.sync_copy(disp_hbm.at[pl.ds(g * _GSZ, _GSZ)], disp_v)
        for c in range(_C):
            pltpu.sync_copy(
                im_hbm.at[pl.ds(im_base + c * (_GROUPS_PER_B * _GSZ), _GSZ)],
                im_v.at[pl.ds(c * _GSZ, _GSZ)])

        def acc_body(xi, xf0):
            for u in range(4):
                x = xi * 4 + u
                xf = xf0 + float(u)
                gidx = lane_w + x
                d = plsc.load_gather(disp_v, [gidx])
                t = xf - d
                tr = (t + _MAGIC) - _MAGIC
                valid = tr >= 0.0
                ti = jnp.maximum(tr.astype(jnp.int32), 0)
                w = jnp.exp(d * _LN_BASE)
                didx = ti * _LANES + lane
                plsc.addupdate_scatter(macc, [didx], w, mask=valid)
                plsc.addupdate_scatter(oacc, [didx], ones, mask=valid)
                for c in range(_C):
                    v = plsc.load_gather(im_v, [gidx + c * _GSZ]) * w
                    plsc.addupdate_scatter(racc, [didx + c * _GSZ], v,
                                           mask=valid)
            return xf0 + 4.0

        lax.fori_loop(0, _W // 4, acc_body, jnp.zeros((_LANES,), jnp.float32))

        def post_body(ti_, _):
            for u in range(4):
                t = ti_ * 4 + u
                base = t * _LANES
                m = macc[pl.ds(base, _LANES)]
                o = oacc[pl.ds(base, _LANES)]
                macc[pl.ds(base, _LANES)] = zeros
                oacc[pl.ds(base, _LANES)] = zeros
                inv = 1.0 / jnp.maximum(m, _EPS)
                out_idx = lane_w + t
                plsc.store_scatter(disp_v, [out_idx],
                                   1.0 - jnp.minimum(o, 1.0))
                for c in range(_C):
                    r = racc[pl.ds(c * _GSZ + base, _LANES)]
                    racc[pl.ds(c * _GSZ + base, _LANES)] = zeros
                    plsc.store_scatter(im_v, [out_idx + c * _GSZ], r * inv)
            return 0

        lax.fori_loop(0, _W // 4, post_body, 0)

        pltpu.sync_copy(disp_v, occ_hbm.at[pl.ds(g * _GSZ, _GSZ)])
        for c in range(_C):
            pltpu.sync_copy(
                im_v.at[pl.ds(c * _GSZ, _GSZ)],
                res_hbm.at[pl.ds(im_base + c * (_GROUPS_PER_B * _GSZ), _GSZ)])
        return 0

    lax.fori_loop(0, _GPW, group_body, 0)


_warp = functools.partial(
    pl.kernel,
    out_type=[
        jax.ShapeDtypeStruct((_B * _C * _H * _W,), jnp.float32),
        jax.ShapeDtypeStruct((_B * _H * _W,), jnp.float32),
    ],
    mesh=plsc.VectorSubcoreMesh(core_axis_name="c", subcore_axis_name="s"),
    compiler_params=pltpu.CompilerParams(needs_layout_passes=False),
    scratch_types=[
        pltpu.VMEM((_GSZ,), jnp.float32),
        pltpu.VMEM((_IMSZ,), jnp.float32),
        pltpu.VMEM((_IMSZ,), jnp.float32),
        pltpu.VMEM((_GSZ,), jnp.float32),
        pltpu.VMEM((_GSZ,), jnp.float32),
    ],
)(_warp_body)


def kernel(im, disp):
    res_flat, occ_flat = _warp(im.reshape(-1), disp.reshape(-1))
    return (res_flat.reshape(_B, _C, _H, _W),
            occ_flat.reshape(_B, 1, _H, _W))


# speedup vs baseline: 47.2322x; 1.6419x over previous
import functools

import jax
import jax.numpy as jnp
import numpy as np
from jax import lax
from jax.experimental import pallas as pl
from jax.experimental.pallas import tpu as pltpu
from jax.experimental.pallas import tpu_sc as plsc

_B, _C, _H, _W = 8, 3, 384, 512
_LANES = 16
_GROUPS = (_B * _H) // _LANES
_GROUPS_PER_B = _H // _LANES
_GSZ = _W * _LANES
_IMSZ = _C * _GSZ
_NWORKERS = 32
_GPW = _GROUPS // _NWORKERS
_EPS = 1e-6
_LN_BASE = float(np.log(1.414))
_MAGIC = float(1.5 * (2 ** 23))


def _warp_body(im_hbm, disp_hbm, res_hbm, occ_hbm, disp_v, im_v, racc, macc,
               oacc):
    nc = 2
    wid = lax.axis_index("s") * nc + lax.axis_index("c")

    lane = lax.iota(jnp.int32, _LANES)
    lane_w = lane * _W
    zeros = jnp.zeros((_LANES,), jnp.float32)
    ones = jnp.ones((_LANES,), jnp.float32)

    def zero_body(i, _):
        for u in range(4):
            b = (i * 4 + u) * _LANES
            macc[pl.ds(b, _LANES)] = zeros
            oacc[pl.ds(b, _LANES)] = zeros
            for c in range(_C):
                racc[pl.ds(c * _GSZ + b, _LANES)] = zeros
        return 0

    lax.fori_loop(0, _W // 4, zero_body, 0)

    def group_body(k, _):
        g = wid * _GPW + k
        b = g // _GROUPS_PER_B
        im_base = b * (_C * _GROUPS_PER_B * _GSZ) + (g % _GROUPS_PER_B) * _GSZ

        pltpu.sync_copy(disp_hbm.at[pl.ds(g * _GSZ, _GSZ)], disp_v)
        for c in range(_C):
            pltpu.sync_copy(
                im_hbm.at[pl.ds(im_base + c * (_GROUPS_PER_B * _GSZ), _GSZ)],
                im_v.at[pl.ds(c * _GSZ, _GSZ)])

        def acc_body(xi, _):
            for u in range(4):
                x = xi * 4 + u
                xcol = (lane + x) & (_W - 1)
                gidx = lane_w + xcol
                d = plsc.load_gather(disp_v, [gidx])
                xf = xcol.astype(jnp.float32)
                tr = ((xf - d) + _MAGIC) - _MAGIC
                valid = tr >= 0.0
                ti = jnp.maximum(tr.astype(jnp.int32), 0)
                w = jnp.exp(d * _LN_BASE)
                didx = ti * _LANES + lane
                plsc.addupdate_scatter(macc, [didx], w, mask=valid)
                plsc.addupdate_scatter(oacc, [didx], ones, mask=valid)
                for c in range(_C):
                    v = plsc.load_gather(im_v, [gidx + c * _GSZ]) * w
                    plsc.addupdate_scatter(racc, [didx + c * _GSZ], v,
                                           mask=valid)
            return 0

        lax.fori_loop(0, _W // 4, acc_body, 0)

        def post_body(ti_, _):
            for u in range(4):
                t = ti_ * 4 + u
                tcol = (lane + t) & (_W - 1)
                aidx = tcol * _LANES + lane
                m = plsc.load_gather(macc, [aidx])
                o = plsc.load_gather(oacc, [aidx])
                plsc.store_scatter(macc, [aidx], zeros)
                plsc.store_scatter(oacc, [aidx], zeros)
                inv = 1.0 / jnp.maximum(m, _EPS)
                out_idx = lane_w + tcol
                plsc.store_scatter(disp_v, [out_idx],
                                   1.0 - jnp.minimum(o, 1.0))
                for c in range(_C):
                    r = plsc.load_gather(racc, [aidx + c * _GSZ])
                    plsc.store_scatter(racc, [aidx + c * _GSZ], zeros)
                    plsc.store_scatter(im_v, [out_idx + c * _GSZ], r * inv)
            return 0

        lax.fori_loop(0, _W // 4, post_body, 0)

        pltpu.sync_copy(disp_v, occ_hbm.at[pl.ds(g * _GSZ, _GSZ)])
        for c in range(_C):
            pltpu.sync_copy(
                im_v.at[pl.ds(c * _GSZ, _GSZ)],
                res_hbm.at[pl.ds(im_base + c * (_GROUPS_PER_B * _GSZ), _GSZ)])
        return 0

    lax.fori_loop(0, _GPW, group_body, 0)


_warp = functools.partial(
    pl.kernel,
    out_type=[
        jax.ShapeDtypeStruct((_B * _C * _H * _W,), jnp.float32),
        jax.ShapeDtypeStruct((_B * _H * _W,), jnp.float32),
    ],
    mesh=plsc.VectorSubcoreMesh(core_axis_name="c", subcore_axis_name="s"),
    compiler_params=pltpu.CompilerParams(needs_layout_passes=False),
    scratch_types=[
        pltpu.VMEM((_GSZ,), jnp.float32),
        pltpu.VMEM((_IMSZ,), jnp.float32),
        pltpu.VMEM((_IMSZ,), jnp.float32),
        pltpu.VMEM((_GSZ,), jnp.float32),
        pltpu.VMEM((_GSZ,), jnp.float32),
    ],
)(_warp_body)


def kernel(im, disp):
    res_flat, occ_flat = _warp(im.reshape(-1), disp.reshape(-1))
    return (res_flat.reshape(_B, _C, _H, _W),
            occ_flat.reshape(_B, 1, _H, _W))
